# Initial kernel scaffold; baseline (speedup 1.0000x reference)
import jax
import jax.numpy as jnp
from jax.experimental import pallas as pl


def kernel(r, edge_index, atom_type_indices, density_table, density_deriv_table, pair_table, pair_deriv_table, embed_table, embed_deriv_table, embed_rho_min, embed_inv_drho):
    raise NotImplementedError("write your pallas kernel here")


# speedup vs baseline: 326.2665x; 326.2665x over previous
import functools

import jax
import jax.numpy as jnp
from jax import lax
from jax.experimental import pallas as pl
from jax.experimental.pallas import tpu as pltpu
from jax.experimental.pallas import tpu_sc as plsc

N_ATOMS = 100000
N_PAIRS = 3200000
E_TYPES = 3
N_R = 8192
N_RHO = 4096
R_MAX = 6.0
INV_DR = (N_R - 1) / R_MAX
RMAX_C = R_MAX * (1.0 - 1e-07)
RHO_CLIP_HI = N_RHO - 1 - 1e-04

NC = 2
NS = 16
NW = NC * NS
L = 16

NA_PAD = 100352
AT_W = NA_PAD // NW
AT_ROWS = AT_W // L
AT_SC = NA_PAD // NS

PAIRS_W = N_PAIRS // NW
CHUNK = 800
VECS = CHUNK // L
NCHUNK = PAIRS_W // CHUNK

DENS_N = E_TYPES * N_R
PAIR_N = E_TYPES * E_TYPES * N_R

OUT_LEN = N_ATOMS + N_PAIRS
ECHUNKS = N_ATOMS // CHUNK

f32 = jnp.float32
i32 = jnp.int32


def _rbin(rr):
    rc = jnp.minimum(jnp.maximum(rr, 0.0), RMAX_C)
    idxf = rc * INV_DR
    idx = idxf.astype(i32)
    frac = idxf - idx.astype(f32)
    nidx = jnp.minimum(idx + 1, N_R - 1)
    return idx, nidx, frac


def _lerp(table, base, idx, nidx, frac):
    v0 = plsc.load_gather(table, [base + idx])
    v1 = plsc.load_gather(table, [base + nidx])
    return v0 + frac * (v1 - v0)


def _vloop(n, body):
    def wrap(i, carry):
        body(i)
        return carry
    lax.fori_loop(0, n, wrap, 0)


def _zero_fill(ref, nwords):
    zeros16 = jnp.zeros((L,), f32)

    def zf(i):
        ref[pl.ds(i * L, L)] = zeros16

    _vloop(nwords // L, zf)


def _pair_pass1_body(r1, s1, d1, tpad, dflat, pflat, rho_out, pe_out,
                     type_s, dens_ts, pair_ts, acc,
                     r_b, s_b, d_b, ti_b, tj_b, i0_b, i1_b, v0_b, v1_b):
    c = lax.axis_index("c")
    s = lax.axis_index("s")
    w = c * NS + s

    sl = pl.ds(s * AT_SC, AT_SC)
    pltpu.sync_copy(tpad.at[sl], type_s.at[sl])
    dsl = pl.ds(s * (DENS_N // NS), DENS_N // NS)
    pltpu.sync_copy(dflat.at[dsl], dens_ts.at[dsl])
    psl = pl.ds(s * (PAIR_N // NS), PAIR_N // NS)
    pltpu.sync_copy(pflat.at[psl], pair_ts.at[psl])
    _zero_fill(acc, NA_PAD)
    plsc.subcore_barrier()

    def chunk1(ci):
        base = w * PAIRS_W + ci * CHUNK
        pltpu.sync_copy(r1.at[pl.ds(base, CHUNK)], r_b)
        pltpu.sync_copy(s1.at[pl.ds(base, CHUNK)], s_b)
        pltpu.sync_copy(d1.at[pl.ds(base, CHUNK)], d_b)
        pltpu.sync_copy(type_s.at[d_b], tj_b)

        def idxrow(i):
            rsl = pl.ds(i * L, L)
            idx, nidx, _ = _rbin(r_b[rsl])
            tj = tj_b[rsl]
            i0_b[rsl] = tj * N_R + idx
            i1_b[rsl] = tj * N_R + nidx

        _vloop(VECS, idxrow)
        pltpu.sync_copy(dens_ts.at[i0_b], v0_b)
        pltpu.sync_copy(dens_ts.at[i1_b], v1_b)

        def accrow(i):
            rsl = pl.ds(i * L, L)
            _, _, frac = _rbin(r_b[rsl])
            v0 = v0_b[rsl]
            dens = v0 + frac * (v1_b[rsl] - v0)
            plsc.addupdate_scatter(acc, [s_b[rsl]], dens)

        _vloop(VECS, accrow)

    _vloop(NCHUNK, chunk1)
    pltpu.sync_copy(acc, rho_out.at[pl.ds(w * NA_PAD, NA_PAD)])
    _zero_fill(acc, NA_PAD)

    def chunk2(ci):
        base = w * PAIRS_W + ci * CHUNK
        pltpu.sync_copy(r1.at[pl.ds(base, CHUNK)], r_b)
        pltpu.sync_copy(s1.at[pl.ds(base, CHUNK)], s_b)
        pltpu.sync_copy(d1.at[pl.ds(base, CHUNK)], d_b)
        pltpu.sync_copy(type_s.at[s_b], ti_b)
        pltpu.sync_copy(type_s.at[d_b], tj_b)

        def idxrow(i):
            rsl = pl.ds(i * L, L)
            idx, nidx, _ = _rbin(r_b[rsl])
            pb = (ti_b[rsl] * E_TYPES + tj_b[rsl]) * N_R
            i0_b[rsl] = pb + idx
            i1_b[rsl] = pb + nidx

        _vloop(VECS, idxrow)
        pltpu.sync_copy(pair_ts.at[i0_b], v0_b)
        pltpu.sync_copy(pair_ts.at[i1_b], v1_b)

        def accrow(i):
            rsl = pl.ds(i * L, L)
            _, _, frac = _rbin(r_b[rsl])
            v0 = v0_b[rsl]
            phi = v0 + frac * (v1_b[rsl] - v0)
            plsc.addupdate_scatter(acc, [s_b[rsl]], 0.5 * phi)

        _vloop(VECS, accrow)

    _vloop(NCHUNK, chunk2)
    pltpu.sync_copy(acc, pe_out.at[pl.ds(w * NA_PAD, NA_PAD)])


def _atom_pass_body(rho_part, pe_part, tpad, eflat, epflat, rmin16, idr16,
                    en_out, fp_out,
                    embed_t, embedp_t, rmin_t, idr_t,
                    rho_b, pe_b, tmp_b, tb, en_b, fp_b):
    c = lax.axis_index("c")
    s = lax.axis_index("s")
    w = c * NS + s
    base = pl.ds(w * AT_W, AT_W)

    pltpu.sync_copy(eflat, embed_t)
    pltpu.sync_copy(epflat, embedp_t)
    pltpu.sync_copy(rmin16, rmin_t)
    pltpu.sync_copy(idr16, idr_t)
    pltpu.sync_copy(tpad.at[base], tb)

    pltpu.sync_copy(rho_part.at[pl.ds(w * AT_W, AT_W)], rho_b)
    pltpu.sync_copy(pe_part.at[pl.ds(w * AT_W, AT_W)], pe_b)

    def red(p):
        pltpu.sync_copy(rho_part.at[pl.ds(p * NA_PAD + w * AT_W, AT_W)],
                        tmp_b)

        def addrow_r(j):
            sl = pl.ds(j * L, L)
            rho_b[sl] = rho_b[sl] + tmp_b[sl]

        _vloop(AT_ROWS, addrow_r)
        pltpu.sync_copy(pe_part.at[pl.ds(p * NA_PAD + w * AT_W, AT_W)],
                        tmp_b)

        def addrow_p(j):
            sl = pl.ds(j * L, L)
            pe_b[sl] = pe_b[sl] + tmp_b[sl]

        _vloop(AT_ROWS, addrow_p)

    def redwrap(p, carry):
        red(p + 1)
        return carry

    lax.fori_loop(0, NW - 1, redwrap, 0)

    def row(j):
        sl = pl.ds(j * L, L)
        t = tb[sl]
        rho = rho_b[sl]
        rm = plsc.load_gather(rmin_t, [t])
        iv = plsc.load_gather(idr_t, [t])
        idxf = jnp.minimum(jnp.maximum((rho - rm) * iv, 0.0), RHO_CLIP_HI)
        idx = idxf.astype(i32)
        frac = idxf - idx.astype(f32)
        nidx = jnp.minimum(idx + 1, N_RHO - 1)
        eb = t * N_RHO
        Fe = _lerp(embed_t, eb, idx, nidx, frac)
        Fp = _lerp(embedp_t, eb, idx, nidx, frac)
        en_b[sl] = Fe + pe_b[sl]
        fp_b[sl] = Fp

    _vloop(AT_ROWS, row)
    pltpu.sync_copy(en_b, en_out.at[base])
    pltpu.sync_copy(fp_b, fp_out.at[base])


def _pair_pass2_body(r1, s1, d1, tpad, fp_pad, dpflat, ppflat, en_pad, out1,
                     type_s, fp_s, densp_t, pairp_t,
                     r_b, s_b, d_b, ti_b, tj_b, fs_b, fd_b, f_b):
    c = lax.axis_index("c")
    s = lax.axis_index("s")
    w = c * NS + s

    sl = pl.ds(s * AT_SC, AT_SC)
    pltpu.sync_copy(tpad.at[sl], type_s.at[sl])
    pltpu.sync_copy(fp_pad.at[sl], fp_s.at[sl])
    pltpu.sync_copy(dpflat, densp_t)
    pltpu.sync_copy(ppflat, pairp_t)

    nch = ECHUNKS // NW + jnp.where(w < ECHUNKS % NW, 1, 0)

    def ecopy(k):
        ebase = (w + k * NW) * CHUNK
        pltpu.sync_copy(en_pad.at[pl.ds(ebase, CHUNK)], f_b)
        pltpu.sync_copy(f_b, out1.at[pl.ds(ebase, CHUNK)])

    _vloop(nch, ecopy)
    plsc.subcore_barrier()

    def chunk(ci):
        base = w * PAIRS_W + ci * CHUNK
        pltpu.sync_copy(r1.at[pl.ds(base, CHUNK)], r_b)
        pltpu.sync_copy(s1.at[pl.ds(base, CHUNK)], s_b)
        pltpu.sync_copy(d1.at[pl.ds(base, CHUNK)], d_b)
        pltpu.sync_copy(type_s.at[s_b], ti_b)
        pltpu.sync_copy(type_s.at[d_b], tj_b)
        pltpu.sync_copy(fp_s.at[s_b], fs_b)
        pltpu.sync_copy(fp_s.at[d_b], fd_b)

        def row(i):
            rsl = pl.ds(i * L, L)
            idx, nidx, frac = _rbin(r_b[rsl])
            ti = ti_b[rsl]
            tj = tj_b[rsl]
            phip = _lerp(pairp_t, (ti * E_TYPES + tj) * N_R, idx, nidx, frac)
            rhop_j = _lerp(densp_t, tj * N_R, idx, nidx, frac)
            rhop_i = _lerp(densp_t, ti * N_R, idx, nidx, frac)
            f_b[rsl] = phip + fs_b[rsl] * rhop_j + fd_b[rsl] * rhop_i

        _vloop(VECS, row)
        pltpu.sync_copy(f_b, out1.at[pl.ds(N_ATOMS + base, CHUNK)])

    _vloop(NCHUNK, chunk)


@functools.cache
def _build(interpret=False):
    def mesh():
        return plsc.VectorSubcoreMesh(core_axis_name="c",
                                      subcore_axis_name="s")

    params = pltpu.CompilerParams(needs_layout_passes=False)

    pass1 = pl.kernel(
        _pair_pass1_body,
        out_type=(
            jax.ShapeDtypeStruct((NW * NA_PAD,), f32),
            jax.ShapeDtypeStruct((NW * NA_PAD,), f32),
        ),
        mesh=mesh(),
        interpret=interpret,
        compiler_params=params,
        scratch_types=[
            pltpu.VMEM_SHARED((NA_PAD,), i32),
            pltpu.VMEM_SHARED((DENS_N,), f32),
            pltpu.VMEM_SHARED((PAIR_N,), f32),
            pltpu.VMEM((NA_PAD,), f32),
            pltpu.VMEM((CHUNK,), f32),
            pltpu.VMEM((CHUNK,), i32),
            pltpu.VMEM((CHUNK,), i32),
            pltpu.VMEM((CHUNK,), i32),
            pltpu.VMEM((CHUNK,), i32),
            pltpu.VMEM((CHUNK,), i32),
            pltpu.VMEM((CHUNK,), i32),
            pltpu.VMEM((CHUNK,), f32),
            pltpu.VMEM((CHUNK,), f32),
        ],
    )

    pass_b = pl.kernel(
        _atom_pass_body,
        out_type=(
            jax.ShapeDtypeStruct((NA_PAD,), f32),
            jax.ShapeDtypeStruct((NA_PAD,), f32),
        ),
        mesh=mesh(),
        interpret=interpret,
        compiler_params=params,
        scratch_types=[
            pltpu.VMEM((E_TYPES * N_RHO,), f32),
            pltpu.VMEM((E_TYPES * N_RHO,), f32),
            pltpu.VMEM((L,), f32),
            pltpu.VMEM((L,), f32),
            pltpu.VMEM((AT_W,), f32),
            pltpu.VMEM((AT_W,), f32),
            pltpu.VMEM((AT_W,), f32),
            pltpu.VMEM((AT_W,), i32),
            pltpu.VMEM((AT_W,), f32),
            pltpu.VMEM((AT_W,), f32),
        ],
    )

    pass2 = pl.kernel(
        _pair_pass2_body,
        out_type=jax.ShapeDtypeStruct((OUT_LEN,), f32),
        mesh=mesh(),
        interpret=interpret,
        compiler_params=params,
        scratch_types=[
            pltpu.VMEM_SHARED((NA_PAD,), i32),
            pltpu.VMEM_SHARED((NA_PAD,), f32),
            pltpu.VMEM((DENS_N,), f32),
            pltpu.VMEM((PAIR_N,), f32),
            pltpu.VMEM((CHUNK,), f32),
            pltpu.VMEM((CHUNK,), i32),
            pltpu.VMEM((CHUNK,), i32),
            pltpu.VMEM((CHUNK,), i32),
            pltpu.VMEM((CHUNK,), i32),
            pltpu.VMEM((CHUNK,), f32),
            pltpu.VMEM((CHUNK,), f32),
            pltpu.VMEM((CHUNK,), f32),
        ],
    )
    return pass1, pass_b, pass2


def _run(r, edge_index, atom_type_indices, density_table,
         density_deriv_table, pair_table, pair_deriv_table,
         embed_table, embed_deriv_table, embed_rho_min, embed_inv_drho,
         interpret=False):
    pass1, pass_b, pass2 = _build(interpret)
    src1 = edge_index[0]
    dst1 = edge_index[1]
    tpad = jnp.pad(atom_type_indices, (0, NA_PAD - N_ATOMS))
    rmin16 = jnp.pad(embed_rho_min, (0, L - E_TYPES))
    idr16 = jnp.pad(embed_inv_drho, (0, L - E_TYPES))

    rho_part, pe_part = pass1(
        r, src1, dst1, tpad,
        density_table.reshape(-1), pair_table.reshape(-1))
    en_pad, fp_pad = pass_b(
        rho_part, pe_part, tpad,
        embed_table.reshape(-1), embed_deriv_table.reshape(-1),
        rmin16, idr16)
    return pass2(
        r, src1, dst1, tpad, fp_pad,
        density_deriv_table.reshape(-1), pair_deriv_table.reshape(-1),
        en_pad)


def kernel(r, edge_index, atom_type_indices, density_table,
           density_deriv_table, pair_table, pair_deriv_table,
           embed_table, embed_deriv_table, embed_rho_min, embed_inv_drho):
    return _run(r, edge_index, atom_type_indices, density_table,
                density_deriv_table, pair_table, pair_deriv_table,
                embed_table, embed_deriv_table, embed_rho_min,
                embed_inv_drho)
